# baseline (speedup 1.0000x reference)
import jax
import jax.numpy as jnp
from jax.experimental import pallas as pl

NUM_EXPERTS = 64
TOP_K = 8
BT = 1024


def _router_body(x_ref, w_ref, b_ref, probs_ref, idx_ref):
    f32 = jnp.float32
    logits_n = jnp.dot(x_ref[:], w_ref[:], preferred_element_type=f32)
    logits_t = jnp.transpose(logits_n)
    logits_t = logits_t + b_ref[:]

    m = jnp.max(logits_t, axis=0, keepdims=True)
    e = jnp.exp(logits_t - m)
    s = jnp.sum(e, axis=0, keepdims=True)
    p = e * (1.0 / s)

    probs_ref[:] = jnp.transpose(p)

    iota = jax.lax.broadcasted_iota(jnp.int32, (NUM_EXPERTS, BT), 0)
    work = p
    rows = []
    for _ in range(TOP_K):
        mx = jnp.max(work, axis=0, keepdims=True)
        sel = jnp.where(work == mx, iota, NUM_EXPERTS)
        idx = jnp.min(sel, axis=0, keepdims=True)
        rows.append(idx)
        work = jnp.where(iota == idx, -1.0, work)
    idx_ref[:] = jnp.transpose(jnp.concatenate(rows, axis=0))


def kernel(hidden_states, layer_id, W_router, b_router):
    b, s, d = hidden_states.shape
    n_tokens = b * s
    x = hidden_states.reshape(n_tokens, d)
    bias = b_router.reshape(NUM_EXPERTS, 1)

    probs, idx_t = pl.pallas_call(
        _router_body,
        grid=(n_tokens // BT,),
        in_specs=[
            pl.BlockSpec((BT, d), lambda i: (i, 0)),
            pl.BlockSpec((d, NUM_EXPERTS), lambda i: (0, 0)),
            pl.BlockSpec((NUM_EXPERTS, 1), lambda i: (0, 0)),
        ],
        out_specs=[
            pl.BlockSpec((BT, NUM_EXPERTS), lambda i: (i, 0)),
            pl.BlockSpec((BT, TOP_K), lambda i: (i, 0)),
        ],
        out_shape=[
            jax.ShapeDtypeStruct((n_tokens, NUM_EXPERTS), jnp.float32),
            jax.ShapeDtypeStruct((n_tokens, TOP_K), jnp.int32),
        ],
    )(x, W_router, bias)

    return (
        probs.reshape(b, s, NUM_EXPERTS),
        idx_t.reshape(b, s, TOP_K),
    )


# speedup vs baseline: 1.3389x; 1.3389x over previous
import jax
import jax.numpy as jnp
from jax.experimental import pallas as pl

NUM_EXPERTS = 64
TOP_K = 8
BT = 2048


def _router_body(x_ref, w_ref, b_ref, probs_ref, idx_ref):
    f32 = jnp.float32
    logits_n = jnp.dot(x_ref[:], w_ref[:], preferred_element_type=f32)
    logits_t = jnp.transpose(logits_n)
    logits_t = logits_t + b_ref[:]

    m = jnp.max(logits_t, axis=0, keepdims=True)
    e = jnp.exp(logits_t - m)
    s = jnp.sum(e, axis=0, keepdims=True)
    p = e * (1.0 / s)

    probs_ref[:] = jnp.transpose(p)

    iota = jax.lax.broadcasted_iota(jnp.int32, (NUM_EXPERTS, BT), 0)
    work = p
    rows = []
    for _ in range(TOP_K):
        mx = jnp.max(work, axis=0, keepdims=True)
        sel = jnp.where(work == mx, iota, NUM_EXPERTS)
        idx = jnp.min(sel, axis=0, keepdims=True)
        rows.append(idx)
        work = jnp.where(iota == idx, -1.0, work)
    idx_ref[:] = jnp.concatenate(rows, axis=0)


def kernel(hidden_states, layer_id, W_router, b_router):
    b, s, d = hidden_states.shape
    n_tokens = b * s
    x = hidden_states.reshape(n_tokens, d)
    bias = b_router.reshape(NUM_EXPERTS, 1)

    probs, idx_t = pl.pallas_call(
        _router_body,
        grid=(n_tokens // BT,),
        in_specs=[
            pl.BlockSpec((BT, d), lambda i: (i, 0)),
            pl.BlockSpec((d, NUM_EXPERTS), lambda i: (0, 0)),
            pl.BlockSpec((NUM_EXPERTS, 1), lambda i: (0, 0)),
        ],
        out_specs=[
            pl.BlockSpec((BT, NUM_EXPERTS), lambda i: (i, 0)),
            pl.BlockSpec((TOP_K, BT), lambda i: (0, i)),
        ],
        out_shape=[
            jax.ShapeDtypeStruct((n_tokens, NUM_EXPERTS), jnp.float32),
            jax.ShapeDtypeStruct((TOP_K, n_tokens), jnp.int32),
        ],
    )(x, W_router, bias)

    return (
        probs.reshape(b, s, NUM_EXPERTS),
        idx_t.T.reshape(b, s, TOP_K),
    )


# speedup vs baseline: 1.3861x; 1.0352x over previous
import jax
import jax.numpy as jnp
from jax.experimental import pallas as pl

NUM_EXPERTS = 64
TOP_K = 8
BT = 4096


def _router_body(x_ref, w_ref, b_ref, probs_ref, idx_ref):
    f32 = jnp.float32
    logits_n = jnp.dot(x_ref[:], w_ref[:], preferred_element_type=f32)
    logits_t = jnp.transpose(logits_n)
    logits_t = logits_t + b_ref[:]

    m = jnp.max(logits_t, axis=0, keepdims=True)
    e = jnp.exp(logits_t - m)
    s = jnp.sum(e, axis=0, keepdims=True)
    p = e * (1.0 / s)

    probs_ref[:] = jnp.transpose(p)

    iota = jax.lax.broadcasted_iota(jnp.int32, (NUM_EXPERTS, BT), 0)
    work = p
    rows = []
    for _ in range(TOP_K):
        mx = jnp.max(work, axis=0, keepdims=True)
        sel = jnp.where(work == mx, iota, NUM_EXPERTS)
        idx = jnp.min(sel, axis=0, keepdims=True)
        rows.append(idx)
        work = jnp.where(iota == idx, -1.0, work)
    idx_ref[:] = jnp.concatenate(rows, axis=0)


def kernel(hidden_states, layer_id, W_router, b_router):
    b, s, d = hidden_states.shape
    n_tokens = b * s
    x = hidden_states.reshape(n_tokens, d)
    bias = b_router.reshape(NUM_EXPERTS, 1)

    probs, idx_t = pl.pallas_call(
        _router_body,
        grid=(n_tokens // BT,),
        in_specs=[
            pl.BlockSpec((BT, d), lambda i: (i, 0)),
            pl.BlockSpec((d, NUM_EXPERTS), lambda i: (0, 0)),
            pl.BlockSpec((NUM_EXPERTS, 1), lambda i: (0, 0)),
        ],
        out_specs=[
            pl.BlockSpec((BT, NUM_EXPERTS), lambda i: (i, 0)),
            pl.BlockSpec((TOP_K, BT), lambda i: (0, i)),
        ],
        out_shape=[
            jax.ShapeDtypeStruct((n_tokens, NUM_EXPERTS), jnp.float32),
            jax.ShapeDtypeStruct((TOP_K, n_tokens), jnp.int32),
        ],
    )(x, W_router, bias)

    return (
        probs.reshape(b, s, NUM_EXPERTS),
        idx_t.T.reshape(b, s, TOP_K),
    )
